# baseline (device time: 290171 ns/iter reference)
import jax
import jax.numpy as jnp
from jax import lax
from jax.experimental import pallas as pl
from jax.experimental.pallas import tpu as pltpu

N_DEV = 4
B, SQ, SKV, HQ, DH, DM = 1, 2048, 2048, 32, 128, 1024
HQ_LOC = HQ // N_DEV
SCALE = 0.08838834764831843
QBLK = 512
WINDOW = 128
N_GLOB = 32


def _attn_body(x_ref, wq_ref, k_ref, v_ref, wo_ref, out_ref, q_ref, ctx_ref):
    q_ref[...] = jnp.dot(
        x_ref[...], wq_ref[...], preferred_element_type=jnp.float32
    ).astype(jnp.bfloat16)

    for h in range(HQ_LOC):
        kh = k_ref[h]
        vh = v_ref[h]

        def qblock(qb, _):
            q0 = qb * QBLK
            qh = q_ref[pl.ds(q0, QBLK), h * DH:(h + 1) * DH]
            s = lax.dot_general(
                qh, kh,
                dimension_numbers=(((1,), (1,)), ((), ())),
                preferred_element_type=jnp.float32,
            ) * SCALE
            qi = q0 + lax.broadcasted_iota(jnp.int32, (QBLK, SKV), 0)
            ki = lax.broadcasted_iota(jnp.int32, (QBLK, SKV), 1)
            mask = (jnp.abs(qi - ki) <= WINDOW) | (ki < N_GLOB) | (qi < N_GLOB)
            s = jnp.where(mask, s, -1e9)
            m = jnp.max(s, axis=1, keepdims=True)
            w = jnp.exp(s - m)
            w = w / jnp.sum(w, axis=1, keepdims=True)
            ctx_ref[pl.ds(q0, QBLK), h * DH:(h + 1) * DH] = jnp.dot(
                w.astype(jnp.bfloat16), vh, preferred_element_type=jnp.float32
            ).astype(jnp.bfloat16)
            return 0

        lax.fori_loop(0, SQ // QBLK, qblock, 0)

    out_ref[...] = jnp.dot(
        ctx_ref[...], wo_ref[...], preferred_element_type=jnp.float32
    ).astype(jnp.bfloat16)


def _allreduce_body(p_ref, out_ref, comm_ref, send_sems, recv_sems):
    my_pos = lax.axis_index("i")
    left = (my_pos - 1) % N_DEV
    right = (my_pos + 1) % N_DEV

    barrier_sem = pltpu.get_barrier_semaphore()
    for nbr in [left, right]:
        pl.semaphore_signal(
            barrier_sem, inc=1,
            device_id=(nbr,), device_id_type=pl.DeviceIdType.MESH,
        )
    pl.semaphore_wait(barrier_sem, 2)

    comm_ref[0] = p_ref[...]
    out_ref[...] = p_ref[...].astype(jnp.float32)

    for h in range(N_DEV - 1):
        send_slot = h % 2
        recv_slot = (h + 1) % 2
        rdma = pltpu.make_async_remote_copy(
            src_ref=comm_ref.at[send_slot],
            dst_ref=comm_ref.at[recv_slot],
            send_sem=send_sems.at[send_slot],
            recv_sem=recv_sems.at[recv_slot],
            device_id=(right,),
            device_id_type=pl.DeviceIdType.MESH,
        )
        rdma.start()
        rdma.wait()
        out_ref[...] += comm_ref[recv_slot].astype(jnp.float32)


def kernel(x, Wq, K_ext, V_ext, Wo):
    i = lax.axis_index("i")

    xb = x[0].astype(jnp.bfloat16)
    wq = Wq.astype(jnp.bfloat16)
    wo = Wo.astype(jnp.bfloat16)
    kh = lax.dynamic_slice_in_dim(K_ext[0], i * HQ_LOC, HQ_LOC, axis=1)
    kh = kh.transpose(1, 0, 2).astype(jnp.bfloat16)
    vh = lax.dynamic_slice_in_dim(V_ext[0], i * HQ_LOC, HQ_LOC, axis=1)
    vh = vh.transpose(1, 0, 2).astype(jnp.bfloat16)

    partial = pl.pallas_call(
        _attn_body,
        out_shape=jax.ShapeDtypeStruct((SQ, DM), jnp.bfloat16),
        in_specs=[pl.BlockSpec(memory_space=pltpu.VMEM)] * 5,
        out_specs=pl.BlockSpec(memory_space=pltpu.VMEM),
        scratch_shapes=[
            pltpu.VMEM((SQ, HQ_LOC * DH), jnp.bfloat16),
            pltpu.VMEM((SQ, HQ_LOC * DH), jnp.bfloat16),
        ],
    )(xb, wq, kh, vh, wo)

    out = pl.pallas_call(
        _allreduce_body,
        out_shape=jax.ShapeDtypeStruct((SQ, DM), jnp.float32),
        in_specs=[pl.BlockSpec(memory_space=pltpu.VMEM)],
        out_specs=pl.BlockSpec(memory_space=pltpu.VMEM),
        scratch_shapes=[
            pltpu.VMEM((2, SQ, DM), jnp.bfloat16),
            pltpu.SemaphoreType.DMA((2,)),
            pltpu.SemaphoreType.DMA((2,)),
        ],
        compiler_params=pltpu.CompilerParams(collective_id=0),
    )(partial)

    return out[None, :, :]


# device time: 114072 ns/iter; 2.5438x vs baseline; 2.5438x over previous
import jax
import jax.numpy as jnp
from jax import lax
from jax.experimental import pallas as pl
from jax.experimental.pallas import tpu as pltpu

N_DEV = 4
B, SQ, SKV, HQ, DH, DM = 1, 2048, 2048, 32, 128, 1024
HQ_LOC = HQ // N_DEV
SCALE = 0.08838834764831843
QBLK = 512
SLAB = 1024
GW = 128
WINDOW = 128
N_GLOB = 32
NEG = -30.0


def _attn_body(x_ref, wq_ref, k_ref, v_ref, wo_ref, out_ref, q_ref, ctx_ref):
    q_ref[...] = (
        jnp.dot(x_ref[...], wq_ref[...], preferred_element_type=jnp.float32)
        * SCALE
    ).astype(jnp.bfloat16)

    for qb in range(SQ // QBLK):
        q0 = qb * QBLK
        start = min(max(q0 - 256, 0), SKV - SLAB)

        qi_l = q0 + lax.broadcasted_iota(jnp.int32, (QBLK, SLAB), 0)
        ki_l = start + lax.broadcasted_iota(jnp.int32, (QBLK, SLAB), 1)
        ok_l = (jnp.abs(qi_l - ki_l) <= WINDOW) & (ki_l >= N_GLOB) \
            & (qi_l >= N_GLOB)
        bias_l = jnp.where(ok_l, 0.0, NEG).astype(jnp.float32)
        qi_g = q0 + lax.broadcasted_iota(jnp.int32, (QBLK, GW), 0)
        ki_g = lax.broadcasted_iota(jnp.int32, (QBLK, GW), 1)
        ok_g = (ki_g < N_GLOB) & (qi_g >= N_GLOB)
        bias_g = jnp.where(ok_g, 0.0, NEG).astype(jnp.float32)

        for h in range(HQ_LOC):
            hc = slice(h * DH, (h + 1) * DH)
            qh = q_ref[q0:q0 + QBLK, hc]
            k_loc = k_ref[start:start + SLAB, hc]
            k_glo = k_ref[0:GW, hc]
            s_l = lax.dot_general(
                qh, k_loc, dimension_numbers=(((1,), (1,)), ((), ())),
                preferred_element_type=jnp.float32,
            ) + bias_l
            s_g = lax.dot_general(
                qh, k_glo, dimension_numbers=(((1,), (1,)), ((), ())),
                preferred_element_type=jnp.float32,
            ) + bias_g
            e_l = jnp.exp(s_l)
            e_g = jnp.exp(s_g)
            r = 1.0 / (
                jnp.sum(e_l, axis=1, keepdims=True)
                + jnp.sum(e_g, axis=1, keepdims=True)
            )
            w_l = (e_l * r).astype(jnp.bfloat16)
            w_g = (e_g * r).astype(jnp.bfloat16)
            ctx = jnp.dot(
                w_l, v_ref[start:start + SLAB, hc],
                preferred_element_type=jnp.float32,
            ) + jnp.dot(
                w_g, v_ref[0:GW, hc], preferred_element_type=jnp.float32
            )
            ctx_ref[q0:q0 + QBLK, hc] = ctx.astype(jnp.bfloat16)

    for h in range(HQ_LOC):
        hc = slice(h * DH, (h + 1) * DH)
        qh = q_ref[0:N_GLOB, hc]
        s = lax.dot_general(
            qh, k_ref[:, hc], dimension_numbers=(((1,), (1,)), ((), ())),
            preferred_element_type=jnp.float32,
        )
        e = jnp.exp(s)
        w = (e / jnp.sum(e, axis=1, keepdims=True)).astype(jnp.bfloat16)
        ctx_ref[0:N_GLOB, hc] = jnp.dot(
            w, v_ref[:, hc], preferred_element_type=jnp.float32
        ).astype(jnp.bfloat16)

    out_ref[...] = jnp.dot(
        ctx_ref[...], wo_ref[...], preferred_element_type=jnp.float32
    ).astype(jnp.bfloat16)


CHUNK = 256


def _allreduce_body(p_ref, out_ref, work_ref, rbuf_ref,
                    s_cw, r_cw, s_ccw, r_ccw):
    my = lax.axis_index("i")
    left = (my - 1) % N_DEV
    right = (my + 1) % N_DEV

    barrier_sem = pltpu.get_barrier_semaphore()
    for nbr in [left, right]:
        pl.semaphore_signal(
            barrier_sem, inc=1,
            device_id=(nbr,), device_id_type=pl.DeviceIdType.MESH,
        )
    pl.semaphore_wait(barrier_sem, 2)

    work_ref[...] = p_ref[...]

    def rdma(src, dst, ssem, rsem, dev):
        return pltpu.make_async_remote_copy(
            src_ref=src, dst_ref=dst, send_sem=ssem, recv_sem=rsem,
            device_id=(dev,), device_id_type=pl.DeviceIdType.MESH,
        )

    for s in range(N_DEV - 1):
        slot = s % 2
        cw_off = ((my - s) % N_DEV) * CHUNK
        ccw_off = N_DEV * CHUNK + ((my + s) % N_DEV) * CHUNK
        cw = rdma(work_ref.at[pl.ds(cw_off, CHUNK)], rbuf_ref.at[0, slot],
                  s_cw.at[slot], r_cw.at[slot], right)
        ccw = rdma(work_ref.at[pl.ds(ccw_off, CHUNK)], rbuf_ref.at[1, slot],
                   s_ccw.at[slot], r_ccw.at[slot], left)
        cw.start()
        ccw.start()
        cw.wait()
        ccw.wait()
        rcw_off = ((my - s - 1) % N_DEV) * CHUNK
        rccw_off = N_DEV * CHUNK + ((my + s + 1) % N_DEV) * CHUNK
        work_ref[pl.ds(rcw_off, CHUNK), :] = (
            work_ref[pl.ds(rcw_off, CHUNK), :].astype(jnp.float32)
            + rbuf_ref[0, slot].astype(jnp.float32)
        ).astype(jnp.bfloat16)
        work_ref[pl.ds(rccw_off, CHUNK), :] = (
            work_ref[pl.ds(rccw_off, CHUNK), :].astype(jnp.float32)
            + rbuf_ref[1, slot].astype(jnp.float32)
        ).astype(jnp.bfloat16)

    for s in range(N_DEV - 1):
        slot = (N_DEV - 1 + s) % 2
        cw_off = ((my + 1 - s) % N_DEV) * CHUNK
        ccw_off = N_DEV * CHUNK + ((my - 1 + s) % N_DEV) * CHUNK
        cw = rdma(work_ref.at[pl.ds(cw_off, CHUNK)],
                  work_ref.at[pl.ds(cw_off, CHUNK)],
                  s_cw.at[slot], r_cw.at[slot], right)
        ccw = rdma(work_ref.at[pl.ds(ccw_off, CHUNK)],
                   work_ref.at[pl.ds(ccw_off, CHUNK)],
                   s_ccw.at[slot], r_ccw.at[slot], left)
        cw.start()
        ccw.start()
        cw.wait()
        ccw.wait()

    out_ref[...] = work_ref[...].astype(jnp.float32)


def kernel(x, Wq, K_ext, V_ext, Wo):
    i = lax.axis_index("i")

    xb = x[0].astype(jnp.bfloat16)
    wq = Wq.astype(jnp.bfloat16)
    wo = Wo.astype(jnp.bfloat16)
    kh = lax.dynamic_slice_in_dim(K_ext[0], i * HQ_LOC, HQ_LOC, axis=1)
    kh = kh.reshape(SKV, HQ_LOC * DH).astype(jnp.bfloat16)
    vh = lax.dynamic_slice_in_dim(V_ext[0], i * HQ_LOC, HQ_LOC, axis=1)
    vh = vh.reshape(SKV, HQ_LOC * DH).astype(jnp.bfloat16)

    partial = pl.pallas_call(
        _attn_body,
        out_shape=jax.ShapeDtypeStruct((SQ, DM), jnp.bfloat16),
        in_specs=[pl.BlockSpec(memory_space=pltpu.VMEM)] * 5,
        out_specs=pl.BlockSpec(memory_space=pltpu.VMEM),
        scratch_shapes=[
            pltpu.VMEM((SQ, HQ_LOC * DH), jnp.bfloat16),
            pltpu.VMEM((SQ, HQ_LOC * DH), jnp.bfloat16),
        ],
    )(xb, wq, kh, vh, wo)

    out = pl.pallas_call(
        _allreduce_body,
        out_shape=jax.ShapeDtypeStruct((SQ, DM), jnp.float32),
        in_specs=[pl.BlockSpec(memory_space=pltpu.VMEM)],
        out_specs=pl.BlockSpec(memory_space=pltpu.VMEM),
        scratch_shapes=[
            pltpu.VMEM((SQ, DM), jnp.bfloat16),
            pltpu.VMEM((2, 2, CHUNK, DM), jnp.bfloat16),
            pltpu.SemaphoreType.DMA((2,)),
            pltpu.SemaphoreType.DMA((2,)),
            pltpu.SemaphoreType.DMA((2,)),
            pltpu.SemaphoreType.DMA((2,)),
        ],
        compiler_params=pltpu.CompilerParams(collective_id=0),
    )(partial)

    return out[None, :, :]


# device time: 101722 ns/iter; 2.8526x vs baseline; 1.1214x over previous
import jax
import jax.numpy as jnp
from jax import lax
from jax.experimental import pallas as pl
from jax.experimental.pallas import tpu as pltpu

N_DEV = 4
B, SQ, SKV, HQ, DH, DM = 1, 2048, 2048, 32, 128, 1024
HQ_LOC = HQ // N_DEV
SCALE = 0.08838834764831843
QBLK = 256
SLAB = 512
GW = 128
WINDOW = 128
N_GLOB = 32
NEG = -30.0
N_CHUNK = SQ // QBLK


def _fused_body(x_ref, wq_ref, k_ref, v_ref, wo_ref, out_ref,
                q_ref, ctx_ref, fix_ref, work_ref, rbuf_ref,
                s_cw, r_cw, s_ccw, r_ccw):
    my = lax.axis_index("i")
    left = (my - 1) % N_DEV
    right = (my + 1) % N_DEV

    barrier_sem = pltpu.get_barrier_semaphore()
    for nbr in [left, right]:
        pl.semaphore_signal(
            barrier_sem, inc=1,
            device_id=(nbr,), device_id_type=pl.DeviceIdType.MESH,
        )
    pl.semaphore_wait(barrier_sem, 2)

    q_ref[...] = (
        jnp.dot(x_ref[...], wq_ref[...], preferred_element_type=jnp.float32)
        * SCALE
    ).astype(jnp.bfloat16)

    for h in range(HQ_LOC):
        hc = slice(h * DH, (h + 1) * DH)
        s = lax.dot_general(
            q_ref[0:N_GLOB, hc], k_ref[:, hc],
            dimension_numbers=(((1,), (1,)), ((), ())),
            preferred_element_type=jnp.float32,
        )
        e = jnp.exp(s)
        w = (e / jnp.sum(e, axis=1, keepdims=True)).astype(jnp.bfloat16)
        fix_ref[:, hc] = jnp.dot(
            w, v_ref[:, hc], preferred_element_type=jnp.float32
        ).astype(jnp.bfloat16)

    def compute_chunk(c):
        q0 = pl.multiple_of(jnp.int32(c * QBLK), QBLK)
        start = pl.multiple_of(
            jnp.clip(q0 - WINDOW, 0, SKV - SLAB), WINDOW
        )

        qi_l = q0 + lax.broadcasted_iota(jnp.int32, (QBLK, SLAB), 0)
        ki_l = start + lax.broadcasted_iota(jnp.int32, (QBLK, SLAB), 1)
        ok_l = (jnp.abs(qi_l - ki_l) <= WINDOW) & (ki_l >= N_GLOB) \
            & (qi_l >= N_GLOB)
        bias_l = jnp.where(ok_l, 0.0, NEG).astype(jnp.float32)
        qi_g = q0 + lax.broadcasted_iota(jnp.int32, (QBLK, GW), 0)
        ki_g = lax.broadcasted_iota(jnp.int32, (QBLK, GW), 1)
        ok_g = (ki_g < N_GLOB) & (qi_g >= N_GLOB)
        bias_g = jnp.where(ok_g, 0.0, NEG).astype(jnp.float32)

        for h in range(HQ_LOC):
            hc = slice(h * DH, (h + 1) * DH)
            qh = q_ref[pl.ds(q0, QBLK), hc]
            s_l = lax.dot_general(
                qh, k_ref[pl.ds(start, SLAB), hc],
                dimension_numbers=(((1,), (1,)), ((), ())),
                preferred_element_type=jnp.float32,
            ) + bias_l
            s_g = lax.dot_general(
                qh, k_ref[0:GW, hc],
                dimension_numbers=(((1,), (1,)), ((), ())),
                preferred_element_type=jnp.float32,
            ) + bias_g
            e_l = jnp.exp(s_l)
            e_g = jnp.exp(s_g)
            r = 1.0 / (
                jnp.sum(e_l, axis=1, keepdims=True)
                + jnp.sum(e_g, axis=1, keepdims=True)
            )
            ctx = jnp.dot(
                (e_l * r).astype(jnp.bfloat16), v_ref[pl.ds(start, SLAB), hc],
                preferred_element_type=jnp.float32,
            ) + jnp.dot(
                (e_g * r).astype(jnp.bfloat16), v_ref[0:GW, hc],
                preferred_element_type=jnp.float32,
            )
            ctx_ref[:, hc] = ctx.astype(jnp.bfloat16)

        @pl.when(q0 == 0)
        def _():
            ctx_ref[0:N_GLOB, :] = fix_ref[...]

        work_ref[pl.ds(q0, QBLK), :] = jnp.dot(
            ctx_ref[...], wo_ref[...], preferred_element_type=jnp.float32
        ).astype(jnp.bfloat16)

    def rdma(src, dst, ssem, rsem, dev):
        return pltpu.make_async_remote_copy(
            src_ref=src, dst_ref=dst, send_sem=ssem, recv_sem=rsem,
            device_id=(dev,), device_id_type=pl.DeviceIdType.MESH,
        )

    compute_chunk(my)
    compute_chunk(N_DEV + (my % N_DEV))

    for s in range(N_DEV - 1):
        slot = s % 2
        cw_off = pl.multiple_of(((my - s) % N_DEV) * QBLK, QBLK)
        ccw_off = pl.multiple_of(
            (N_DEV + (my + s) % N_DEV) * QBLK, QBLK)
        cw = rdma(work_ref.at[pl.ds(cw_off, QBLK)], rbuf_ref.at[0, slot],
                  s_cw.at[slot], r_cw.at[slot], right)
        ccw = rdma(work_ref.at[pl.ds(ccw_off, QBLK)], rbuf_ref.at[1, slot],
                   s_ccw.at[slot], r_ccw.at[slot], left)
        cw.start()
        ccw.start()
        if s < N_DEV - 2:
            compute_chunk((my - s - 1) % N_DEV)
            compute_chunk(N_DEV + (my + s + 1) % N_DEV)
        else:
            compute_chunk((my + 1) % N_DEV)
            compute_chunk(N_DEV + (my - 1) % N_DEV)
        cw.wait()
        ccw.wait()
        rcw_off = pl.multiple_of(((my - s - 1) % N_DEV) * QBLK, QBLK)
        rccw_off = pl.multiple_of(
            (N_DEV + (my + s + 1) % N_DEV) * QBLK, QBLK)
        work_ref[pl.ds(rcw_off, QBLK), :] = (
            work_ref[pl.ds(rcw_off, QBLK), :].astype(jnp.float32)
            + rbuf_ref[0, slot].astype(jnp.float32)
        ).astype(jnp.bfloat16)
        work_ref[pl.ds(rccw_off, QBLK), :] = (
            work_ref[pl.ds(rccw_off, QBLK), :].astype(jnp.float32)
            + rbuf_ref[1, slot].astype(jnp.float32)
        ).astype(jnp.bfloat16)

    for s in range(N_DEV - 1):
        slot = (N_DEV - 1 + s) % 2
        cw_off = pl.multiple_of(((my + 1 - s) % N_DEV) * QBLK, QBLK)
        ccw_off = pl.multiple_of(
            (N_DEV + (my - 1 + s) % N_DEV) * QBLK, QBLK)
        cw = rdma(work_ref.at[pl.ds(cw_off, QBLK)],
                  work_ref.at[pl.ds(cw_off, QBLK)],
                  s_cw.at[slot], r_cw.at[slot], right)
        ccw = rdma(work_ref.at[pl.ds(ccw_off, QBLK)],
                   work_ref.at[pl.ds(ccw_off, QBLK)],
                   s_ccw.at[slot], r_ccw.at[slot], left)
        cw.start()
        ccw.start()
        cw.wait()
        ccw.wait()

    out_ref[...] = work_ref[...].astype(jnp.float32)


def kernel(x, Wq, K_ext, V_ext, Wo):
    i = lax.axis_index("i")

    xb = x[0].astype(jnp.bfloat16)
    wq = Wq.astype(jnp.bfloat16)
    wo = Wo.astype(jnp.bfloat16)
    kh = lax.dynamic_slice_in_dim(K_ext[0], i * HQ_LOC, HQ_LOC, axis=1)
    kh = kh.reshape(SKV, HQ_LOC * DH).astype(jnp.bfloat16)
    vh = lax.dynamic_slice_in_dim(V_ext[0], i * HQ_LOC, HQ_LOC, axis=1)
    vh = vh.reshape(SKV, HQ_LOC * DH).astype(jnp.bfloat16)

    out = pl.pallas_call(
        _fused_body,
        out_shape=jax.ShapeDtypeStruct((SQ, DM), jnp.float32),
        in_specs=[pl.BlockSpec(memory_space=pltpu.VMEM)] * 5,
        out_specs=pl.BlockSpec(memory_space=pltpu.VMEM),
        scratch_shapes=[
            pltpu.VMEM((SQ, HQ_LOC * DH), jnp.bfloat16),
            pltpu.VMEM((QBLK, HQ_LOC * DH), jnp.bfloat16),
            pltpu.VMEM((N_GLOB, HQ_LOC * DH), jnp.bfloat16),
            pltpu.VMEM((SQ, DM), jnp.bfloat16),
            pltpu.VMEM((2, 2, QBLK, DM), jnp.bfloat16),
            pltpu.SemaphoreType.DMA((2,)),
            pltpu.SemaphoreType.DMA((2,)),
            pltpu.SemaphoreType.DMA((2,)),
            pltpu.SemaphoreType.DMA((2,)),
        ],
        compiler_params=pltpu.CompilerParams(collective_id=0),
    )(xb, wq, kh, vh, wo)

    return out[None, :, :]


# device time: 91845 ns/iter; 3.1594x vs baseline; 1.1075x over previous
import jax
import jax.numpy as jnp
from jax import lax
from jax.experimental import pallas as pl
from jax.experimental.pallas import tpu as pltpu

N_DEV = 4
B, SQ, SKV, HQ, DH, DM = 1, 2048, 2048, 32, 128, 1024
HQ_LOC = HQ // N_DEV
SCALE = 0.08838834764831843
QBLK = 256
SLAB = 512
GW = 128
WINDOW = 128
N_GLOB = 32
NEG = -30.0
N_CHUNK = SQ // QBLK


def _fused_body(x_ref, wq_ref, k_ref, v_ref, wo_ref, work_ref,
                q_ref, ctx_ref, fix_ref, ones_ref, rbuf_ref,
                s_cw, r_cw, s_ccw, r_ccw):
    my = lax.axis_index("i")
    left = (my - 1) % N_DEV
    right = (my + 1) % N_DEV

    barrier_sem = pltpu.get_barrier_semaphore()
    for nbr in [left, right]:
        pl.semaphore_signal(
            barrier_sem, inc=1,
            device_id=(nbr,), device_id_type=pl.DeviceIdType.MESH,
        )
    pl.semaphore_wait(barrier_sem, 2)

    ones_ref[...] = jnp.ones((SKV, DH), jnp.bfloat16)

    q_ref[...] = (
        jnp.dot(x_ref[...], wq_ref[...], preferred_element_type=jnp.float32)
        * SCALE
    ).astype(jnp.bfloat16)

    for h in range(HQ_LOC):
        hc = slice(h * DH, (h + 1) * DH)
        s = lax.dot_general(
            q_ref[0:N_GLOB, hc], k_ref[:, hc],
            dimension_numbers=(((1,), (1,)), ((), ())),
            preferred_element_type=jnp.float32,
        )
        e = jnp.exp(s).astype(jnp.bfloat16)
        ctx_un = jnp.dot(
            e, v_ref[:, hc], preferred_element_type=jnp.float32
        )
        den = jnp.dot(
            e, ones_ref[...], preferred_element_type=jnp.float32
        )
        fix_ref[:, hc] = (ctx_un / den).astype(jnp.bfloat16)

    def compute_chunk(c):
        q0 = pl.multiple_of(jnp.int32(c * QBLK), QBLK)
        start = pl.multiple_of(
            jnp.clip(q0 - WINDOW, 0, SKV - SLAB), WINDOW
        )

        qi_l = q0 + lax.broadcasted_iota(jnp.int32, (QBLK, SLAB), 0)
        ki_l = start + lax.broadcasted_iota(jnp.int32, (QBLK, SLAB), 1)
        ok_l = (jnp.abs(qi_l - ki_l) <= WINDOW) & (ki_l >= N_GLOB) \
            & (qi_l >= N_GLOB)
        bias_l = jnp.where(ok_l, 0.0, NEG).astype(jnp.float32)
        qi_g = q0 + lax.broadcasted_iota(jnp.int32, (QBLK, GW), 0)
        ki_g = lax.broadcasted_iota(jnp.int32, (QBLK, GW), 1)
        ok_g = (ki_g < N_GLOB) & (qi_g >= N_GLOB)
        bias_g = jnp.where(ok_g, 0.0, NEG).astype(jnp.float32)

        for h in range(HQ_LOC):
            hc = slice(h * DH, (h + 1) * DH)
            qh = q_ref[pl.ds(q0, QBLK), hc]
            s_l = lax.dot_general(
                qh, k_ref[pl.ds(start, SLAB), hc],
                dimension_numbers=(((1,), (1,)), ((), ())),
                preferred_element_type=jnp.float32,
            ) + bias_l
            s_g = lax.dot_general(
                qh, k_ref[0:GW, hc],
                dimension_numbers=(((1,), (1,)), ((), ())),
                preferred_element_type=jnp.float32,
            ) + bias_g
            e_l = jnp.exp(s_l).astype(jnp.bfloat16)
            e_g = jnp.exp(s_g).astype(jnp.bfloat16)
            ctx_un = jnp.dot(
                e_l, v_ref[pl.ds(start, SLAB), hc],
                preferred_element_type=jnp.float32,
            ) + jnp.dot(
                e_g, v_ref[0:GW, hc], preferred_element_type=jnp.float32,
            )
            den = jnp.dot(
                e_l, ones_ref[0:SLAB], preferred_element_type=jnp.float32,
            ) + jnp.dot(
                e_g, ones_ref[0:GW], preferred_element_type=jnp.float32,
            )
            ctx_ref[:, hc] = (ctx_un / den).astype(jnp.bfloat16)

        @pl.when(q0 == 0)
        def _():
            ctx_ref[0:N_GLOB, :] = fix_ref[...]

        work_ref[pl.ds(q0, QBLK), :] = jnp.dot(
            ctx_ref[...], wo_ref[...], preferred_element_type=jnp.float32
        ).astype(jnp.bfloat16)

    def rdma(src, dst, ssem, rsem, dev):
        return pltpu.make_async_remote_copy(
            src_ref=src, dst_ref=dst, send_sem=ssem, recv_sem=rsem,
            device_id=(dev,), device_id_type=pl.DeviceIdType.MESH,
        )

    compute_chunk(my)
    compute_chunk(N_DEV + (my % N_DEV))

    for s in range(N_DEV - 1):
        slot = s % 2
        cw_off = pl.multiple_of(((my - s) % N_DEV) * QBLK, QBLK)
        ccw_off = pl.multiple_of(
            (N_DEV + (my + s) % N_DEV) * QBLK, QBLK)
        cw = rdma(work_ref.at[pl.ds(cw_off, QBLK)], rbuf_ref.at[0, slot],
                  s_cw.at[slot], r_cw.at[slot], right)
        ccw = rdma(work_ref.at[pl.ds(ccw_off, QBLK)], rbuf_ref.at[1, slot],
                   s_ccw.at[slot], r_ccw.at[slot], left)
        cw.start()
        ccw.start()
        if s < N_DEV - 2:
            compute_chunk((my - s - 1) % N_DEV)
            compute_chunk(N_DEV + (my + s + 1) % N_DEV)
        else:
            compute_chunk((my + 1) % N_DEV)
            compute_chunk(N_DEV + (my - 1) % N_DEV)
        cw.wait()
        ccw.wait()
        rcw_off = pl.multiple_of(((my - s - 1) % N_DEV) * QBLK, QBLK)
        rccw_off = pl.multiple_of(
            (N_DEV + (my + s + 1) % N_DEV) * QBLK, QBLK)
        work_ref[pl.ds(rcw_off, QBLK), :] = (
            work_ref[pl.ds(rcw_off, QBLK), :].astype(jnp.float32)
            + rbuf_ref[0, slot].astype(jnp.float32)
        ).astype(jnp.bfloat16)
        work_ref[pl.ds(rccw_off, QBLK), :] = (
            work_ref[pl.ds(rccw_off, QBLK), :].astype(jnp.float32)
            + rbuf_ref[1, slot].astype(jnp.float32)
        ).astype(jnp.bfloat16)

    for s in range(N_DEV - 1):
        slot = (N_DEV - 1 + s) % 2
        cw_off = pl.multiple_of(((my + 1 - s) % N_DEV) * QBLK, QBLK)
        ccw_off = pl.multiple_of(
            (N_DEV + (my - 1 + s) % N_DEV) * QBLK, QBLK)
        cw = rdma(work_ref.at[pl.ds(cw_off, QBLK)],
                  work_ref.at[pl.ds(cw_off, QBLK)],
                  s_cw.at[slot], r_cw.at[slot], right)
        ccw = rdma(work_ref.at[pl.ds(ccw_off, QBLK)],
                   work_ref.at[pl.ds(ccw_off, QBLK)],
                   s_ccw.at[slot], r_ccw.at[slot], left)
        cw.start()
        ccw.start()
        cw.wait()
        ccw.wait()


def kernel(x, Wq, K_ext, V_ext, Wo):
    i = lax.axis_index("i")

    xb = x[0].astype(jnp.bfloat16)
    wq = Wq.astype(jnp.bfloat16)
    wo = Wo.astype(jnp.bfloat16)
    kh = lax.dynamic_slice_in_dim(K_ext[0], i * HQ_LOC, HQ_LOC, axis=1)
    kh = kh.reshape(SKV, HQ_LOC * DH).astype(jnp.bfloat16)
    vh = lax.dynamic_slice_in_dim(V_ext[0], i * HQ_LOC, HQ_LOC, axis=1)
    vh = vh.reshape(SKV, HQ_LOC * DH).astype(jnp.bfloat16)

    out = pl.pallas_call(
        _fused_body,
        out_shape=jax.ShapeDtypeStruct((SQ, DM), jnp.bfloat16),
        in_specs=[pl.BlockSpec(memory_space=pltpu.VMEM)] * 5,
        out_specs=pl.BlockSpec(memory_space=pltpu.VMEM),
        scratch_shapes=[
            pltpu.VMEM((SQ, HQ_LOC * DH), jnp.bfloat16),
            pltpu.VMEM((QBLK, HQ_LOC * DH), jnp.bfloat16),
            pltpu.VMEM((N_GLOB, HQ_LOC * DH), jnp.bfloat16),
            pltpu.VMEM((SKV, DH), jnp.bfloat16),
            pltpu.VMEM((2, 2, QBLK, DM), jnp.bfloat16),
            pltpu.SemaphoreType.DMA((2,)),
            pltpu.SemaphoreType.DMA((2,)),
            pltpu.SemaphoreType.DMA((2,)),
            pltpu.SemaphoreType.DMA((2,)),
        ],
        compiler_params=pltpu.CompilerParams(collective_id=0),
    )(xb, wq, kh, vh, wo)

    return out[None, :, :]
